# baseline (speedup 1.0000x reference)
import functools

import jax
import jax.numpy as jnp
from jax import lax
from jax.experimental import pallas as pl
from jax.experimental.pallas import tpu as pltpu
from jax.experimental.pallas import tpu_sc as plsc

B = 16384
D = 128
NC = 2
NS = 16
NW = NC * NS
BPW = B // NW
CH = 128
NCH = BPW // CH

_mesh = plsc.VectorSubcoreMesh(core_axis_name="c", subcore_axis_name="s")


@functools.partial(
    pl.kernel,
    mesh=_mesh,
    compiler_params=pltpu.CompilerParams(needs_layout_passes=False),
    out_type=jax.ShapeDtypeStruct((B,), jnp.float32),
    scratch_types=[
        pltpu.VMEM((CH,), jnp.int32),
        pltpu.VMEM((CH,), jnp.int32),
        pltpu.VMEM((CH, D), jnp.float32),
        pltpu.VMEM((CH, D), jnp.float32),
        pltpu.VMEM((BPW,), jnp.float32),
        pltpu.SemaphoreType.DMA,
        pltpu.SemaphoreType.DMA,
    ],
)
def _sc_dot_kernel(user_hbm, item_hbm, uf_hbm, if_hbm, out_hbm,
                   uidx_v, iidx_v, u_v, v_v, o_v, sem_u, sem_v):
    wid = lax.axis_index("s") * NC + lax.axis_index("c")
    base = wid * BPW
    for c in range(NCH):
        cbase = base + c * CH
        pltpu.sync_copy(user_hbm.at[pl.ds(cbase, CH)], uidx_v)
        pltpu.sync_copy(item_hbm.at[pl.ds(cbase, CH)], iidx_v)
        cu = pltpu.async_copy(uf_hbm.at[uidx_v], u_v, sem_u)
        cv = pltpu.async_copy(if_hbm.at[iidx_v], v_v, sem_v)
        cu.wait()
        cv.wait()

        for g in range(CH // 16):
            rows = g * 16 + lax.iota(jnp.int32, 16)

            @plsc.parallel_loop(0, D, unroll=8,
                                carry=jnp.zeros((16,), jnp.float32))
            def acc_loop(d, acc, rows=rows):
                cols = jnp.full((16,), d, jnp.int32)
                uu = plsc.load_gather(u_v, [rows, cols])
                vv = plsc.load_gather(v_v, [rows, cols])
                return acc + uu * vv

            o_v[pl.ds(c * CH + g * 16, 16)] = acc_loop
    pltpu.sync_copy(o_v, out_hbm.at[pl.ds(base, BPW)])


def kernel(user, item, user_factors, item_factors):
    return _sc_dot_kernel(user.astype(jnp.int32), item.astype(jnp.int32),
                          user_factors, item_factors)


# speedup vs baseline: 2.4194x; 2.4194x over previous
import functools

import jax
import jax.numpy as jnp
from jax import lax
from jax.experimental import pallas as pl
from jax.experimental.pallas import tpu as pltpu
from jax.experimental.pallas import tpu_sc as plsc

B = 16384
D = 128
NC = 2
NS = 16
NW = NC * NS
BPW = B // NW
CH = 128
NCH = BPW // CH

_mesh = plsc.VectorSubcoreMesh(core_axis_name="c", subcore_axis_name="s")


@functools.partial(
    pl.kernel,
    mesh=_mesh,
    compiler_params=pltpu.CompilerParams(needs_layout_passes=False),
    out_type=jax.ShapeDtypeStruct((B,), jnp.float32),
    scratch_types=[
        pltpu.VMEM((CH,), jnp.int32),
        pltpu.VMEM((CH,), jnp.int32),
        pltpu.VMEM((CH, D), jnp.float32),
        pltpu.VMEM((CH, D), jnp.float32),
        pltpu.VMEM((BPW,), jnp.float32),
        pltpu.SemaphoreType.DMA,
        pltpu.SemaphoreType.DMA,
    ],
)
def _sc_dot_kernel(user_hbm, item_hbm, uf_hbm, if_hbm, out_hbm,
                   uidx_v, iidx_v, u_v, v_v, o_v, sem_u, sem_v):
    wid = lax.axis_index("s") * NC + lax.axis_index("c")
    base = wid * BPW
    for c in range(NCH):
        cbase = base + c * CH
        pltpu.sync_copy(user_hbm.at[pl.ds(cbase, CH)], uidx_v)
        pltpu.sync_copy(item_hbm.at[pl.ds(cbase, CH)], iidx_v)
        cu = pltpu.async_copy(uf_hbm.at[uidx_v], u_v, sem_u)
        cv = pltpu.async_copy(if_hbm.at[iidx_v], v_v, sem_v)
        cu.wait()
        cv.wait()

        for g in range(CH // 16):
            rows = g * 16 + lax.iota(jnp.int32, 16)
            lane = lax.iota(jnp.int32, 16)

            @plsc.parallel_loop(0, D, unroll=8,
                                carry=jnp.zeros((16,), jnp.float32))
            def acc_loop(d, acc, rows=rows, lane=lane):
                cols = (d + lane) & (D - 1)
                uu = plsc.load_gather(u_v, [rows, cols])
                vv = plsc.load_gather(v_v, [rows, cols])
                return acc + uu * vv

            o_v[pl.ds(c * CH + g * 16, 16)] = acc_loop
    pltpu.sync_copy(o_v, out_hbm.at[pl.ds(base, BPW)])


def kernel(user, item, user_factors, item_factors):
    return _sc_dot_kernel(user.astype(jnp.int32), item.astype(jnp.int32),
                          user_factors, item_factors)


# speedup vs baseline: 2.8714x; 1.1868x over previous
import functools

import jax
import jax.numpy as jnp
from jax import lax
from jax.experimental import pallas as pl
from jax.experimental.pallas import tpu as pltpu
from jax.experimental.pallas import tpu_sc as plsc

B = 16384
D = 128
NC = 2
NS = 16
NW = NC * NS
BPW = B // NW
CH = 128
NCH = BPW // CH

_mesh = plsc.VectorSubcoreMesh(core_axis_name="c", subcore_axis_name="s")


@functools.partial(
    pl.kernel,
    mesh=_mesh,
    compiler_params=pltpu.CompilerParams(needs_layout_passes=False),
    out_type=jax.ShapeDtypeStruct((B,), jnp.float32),
    scratch_types=[
        pltpu.VMEM((BPW,), jnp.int32),
        pltpu.VMEM((BPW,), jnp.int32),
        pltpu.VMEM((CH, D), jnp.float32),
        pltpu.VMEM((CH, D), jnp.float32),
        pltpu.VMEM((CH, D), jnp.float32),
        pltpu.VMEM((CH, D), jnp.float32),
        pltpu.VMEM((BPW,), jnp.float32),
        pltpu.SemaphoreType.DMA,
        pltpu.SemaphoreType.DMA,
        pltpu.SemaphoreType.DMA,
        pltpu.SemaphoreType.DMA,
    ],
)
def _sc_dot_kernel(user_hbm, item_hbm, uf_hbm, if_hbm, out_hbm,
                   uidx_v, iidx_v, u0_v, u1_v, v0_v, v1_v, o_v,
                   sem_u0, sem_u1, sem_v0, sem_v1):
    wid = lax.axis_index("s") * NC + lax.axis_index("c")
    base = wid * BPW
    u_bufs = (u0_v, u1_v)
    v_bufs = (v0_v, v1_v)
    u_sems = (sem_u0, sem_u1)
    v_sems = (sem_v0, sem_v1)

    pltpu.sync_copy(user_hbm.at[pl.ds(base, BPW)], uidx_v)
    pltpu.sync_copy(item_hbm.at[pl.ds(base, BPW)], iidx_v)

    def issue(c):
        p = c % 2
        cu = pltpu.async_copy(uf_hbm.at[uidx_v.at[pl.ds(c * CH, CH)]],
                              u_bufs[p], u_sems[p])
        cv = pltpu.async_copy(if_hbm.at[iidx_v.at[pl.ds(c * CH, CH)]],
                              v_bufs[p], v_sems[p])
        return cu, cv

    inflight = issue(0)
    lane = lax.iota(jnp.int32, 16)
    for c in range(NCH):
        cu, cv = inflight
        cu.wait()
        cv.wait()
        if c + 1 < NCH:
            inflight = issue(c + 1)
        p = c % 2
        u_v, v_v = u_bufs[p], v_bufs[p]

        for g in range(CH // 16):
            rows = g * 16 + lane

            zero = jnp.zeros((16,), jnp.float32)

            @plsc.parallel_loop(0, D, step=2, unroll=4, carry=(zero, zero))
            def acc_loop(d, carry, rows=rows, lane=lane):
                a0, a1 = carry
                c0 = (d + lane) & (D - 1)
                c1 = (d + 1 + lane) & (D - 1)
                a0 = a0 + plsc.load_gather(u_v, [rows, c0]) * \
                    plsc.load_gather(v_v, [rows, c0])
                a1 = a1 + plsc.load_gather(u_v, [rows, c1]) * \
                    plsc.load_gather(v_v, [rows, c1])
                return a0, a1

            o_v[pl.ds(c * CH + g * 16, 16)] = acc_loop[0] + acc_loop[1]
    pltpu.sync_copy(o_v, out_hbm.at[pl.ds(base, BPW)])


def kernel(user, item, user_factors, item_factors):
    return _sc_dot_kernel(user.astype(jnp.int32), item.astype(jnp.int32),
                          user_factors, item_factors)


# speedup vs baseline: 3.2142x; 1.1194x over previous
import functools

import jax
import jax.numpy as jnp
from jax import lax
from jax.experimental import pallas as pl
from jax.experimental.pallas import tpu as pltpu
from jax.experimental.pallas import tpu_sc as plsc

B = 16384
D = 128
NC = 2
NS = 16
NW = NC * NS
BPW = B // NW
CH = 128
NCH = BPW // CH

_mesh = plsc.VectorSubcoreMesh(core_axis_name="c", subcore_axis_name="s")


@functools.partial(
    pl.kernel,
    mesh=_mesh,
    compiler_params=pltpu.CompilerParams(needs_layout_passes=False),
    out_type=jax.ShapeDtypeStruct((B,), jnp.float32),
    scratch_types=[
        pltpu.VMEM((BPW,), jnp.int32),
        pltpu.VMEM((BPW,), jnp.int32),
        pltpu.VMEM((CH, D), jnp.float32),
        pltpu.VMEM((CH, D), jnp.float32),
        pltpu.VMEM((CH, D), jnp.float32),
        pltpu.VMEM((CH, D), jnp.float32),
        pltpu.VMEM((BPW,), jnp.float32),
        pltpu.SemaphoreType.DMA,
        pltpu.SemaphoreType.DMA,
        pltpu.SemaphoreType.DMA,
        pltpu.SemaphoreType.DMA,
    ],
)
def _sc_dot_kernel(user_hbm, item_hbm, uf_hbm, if_hbm, out_hbm,
                   uidx_v, iidx_v, u0_v, u1_v, v0_v, v1_v, o_v,
                   sem_u0, sem_u1, sem_v0, sem_v1):
    wid = lax.axis_index("s") * NC + lax.axis_index("c")
    base = wid * BPW
    u_bufs = (u0_v, u1_v)
    v_bufs = (v0_v, v1_v)
    u_sems = (sem_u0, sem_u1)
    v_sems = (sem_v0, sem_v1)

    pltpu.sync_copy(user_hbm.at[pl.ds(base, BPW)], uidx_v)
    pltpu.sync_copy(item_hbm.at[pl.ds(base, BPW)], iidx_v)

    def issue(c):
        p = c % 2
        cu = pltpu.async_copy(uf_hbm.at[uidx_v.at[pl.ds(c * CH, CH)]],
                              u_bufs[p], u_sems[p])
        cv = pltpu.async_copy(if_hbm.at[iidx_v.at[pl.ds(c * CH, CH)]],
                              v_bufs[p], v_sems[p])
        return cu, cv

    inflight = issue(0)
    lane = lax.iota(jnp.int32, 16)
    for c in range(NCH):
        cu, cv = inflight
        cu.wait()
        cv.wait()
        if c + 1 < NCH:
            inflight = issue(c + 1)
        p = c % 2
        u_v, v_v = u_bufs[p], v_bufs[p]

        def group_body(g, carry, c=c):
            rows = g * 16 + lane

            zero = jnp.zeros((16,), jnp.float32)

            @plsc.parallel_loop(0, D, step=2, unroll=4,
                                carry=(zero, zero, lane))
            def acc_loop(d, state, rows=rows):
                a0, a1, col = state
                c1 = (col + 1) & (D - 1)
                a0 = a0 + plsc.load_gather(u_v, [rows, col]) * \
                    plsc.load_gather(v_v, [rows, col])
                a1 = a1 + plsc.load_gather(u_v, [rows, c1]) * \
                    plsc.load_gather(v_v, [rows, c1])
                return a0, a1, (col + 2) & (D - 1)

            o_v[pl.ds(c * CH + g * 16, 16)] = acc_loop[0] + acc_loop[1]
            return carry

        lax.fori_loop(0, CH // 16, group_body, 0)
    pltpu.sync_copy(o_v, out_hbm.at[pl.ds(base, BPW)])


def kernel(user, item, user_factors, item_factors):
    return _sc_dot_kernel(user.astype(jnp.int32), item.astype(jnp.int32),
                          user_factors, item_factors)


# speedup vs baseline: 3.2841x; 1.0217x over previous
import functools

import jax
import jax.numpy as jnp
from jax import lax
from jax.experimental import pallas as pl
from jax.experimental.pallas import tpu as pltpu
from jax.experimental.pallas import tpu_sc as plsc

B = 16384
D = 128
NC = 2
NS = 16
NW = NC * NS
BPW = B // NW
CH = 128
NCH = BPW // CH

_mesh = plsc.VectorSubcoreMesh(core_axis_name="c", subcore_axis_name="s")


@functools.partial(
    pl.kernel,
    mesh=_mesh,
    compiler_params=pltpu.CompilerParams(needs_layout_passes=False),
    out_type=jax.ShapeDtypeStruct((B,), jnp.float32),
    scratch_types=[
        pltpu.VMEM((BPW,), jnp.int32),
        pltpu.VMEM((BPW,), jnp.int32),
        pltpu.VMEM((CH, D), jnp.float32),
        pltpu.VMEM((CH, D), jnp.float32),
        pltpu.VMEM((CH, D), jnp.float32),
        pltpu.VMEM((CH, D), jnp.float32),
        pltpu.VMEM((BPW,), jnp.float32),
        pltpu.SemaphoreType.DMA,
        pltpu.SemaphoreType.DMA,
        pltpu.SemaphoreType.DMA,
        pltpu.SemaphoreType.DMA,
    ],
)
def _sc_dot_kernel(user_hbm, item_hbm, uf_hbm, if_hbm, out_hbm,
                   uidx_v, iidx_v, u0_v, u1_v, v0_v, v1_v, o_v,
                   sem_u0, sem_u1, sem_v0, sem_v1):
    wid = lax.axis_index("s") * NC + lax.axis_index("c")
    base = wid * BPW
    u_bufs = (u0_v, u1_v)
    v_bufs = (v0_v, v1_v)
    u_sems = (sem_u0, sem_u1)
    v_sems = (sem_v0, sem_v1)
    lane = lax.iota(jnp.int32, 16)

    pltpu.sync_copy(user_hbm.at[pl.ds(base, BPW)], uidx_v)
    pltpu.sync_copy(item_hbm.at[pl.ds(base, BPW)], iidx_v)

    def issue(c, p):
        pltpu.async_copy(uf_hbm.at[uidx_v.at[pl.ds(c * CH, CH)]],
                         u_bufs[p], u_sems[p])
        pltpu.async_copy(if_hbm.at[iidx_v.at[pl.ds(c * CH, CH)]],
                         v_bufs[p], v_sems[p])

    def drain(c, p):
        pltpu.make_async_copy(uf_hbm.at[uidx_v.at[pl.ds(c * CH, CH)]],
                              u_bufs[p], u_sems[p]).wait()
        pltpu.make_async_copy(if_hbm.at[iidx_v.at[pl.ds(c * CH, CH)]],
                              v_bufs[p], v_sems[p]).wait()

    def compute(c, p):
        u_v, v_v = u_bufs[p], v_bufs[p]

        def group_body(g, carry):
            rows = g * 16 + lane

            zero = jnp.zeros((16,), jnp.float32)

            @plsc.parallel_loop(0, D, step=2, unroll=4,
                                carry=(zero, zero, lane))
            def acc_loop(d, state, rows=rows):
                a0, a1, col = state
                c1 = (col + 1) & (D - 1)
                a0 = a0 + plsc.load_gather(u_v, [rows, col]) * \
                    plsc.load_gather(v_v, [rows, col])
                a1 = a1 + plsc.load_gather(u_v, [rows, c1]) * \
                    plsc.load_gather(v_v, [rows, c1])
                return a0, a1, (col + 2) & (D - 1)

            o_v[pl.ds(c * CH + g * 16, 16)] = acc_loop[0] + acc_loop[1]
            return carry

        lax.fori_loop(0, CH // 16, group_body, 0)

    issue(0, 0)

    def pair_body(pr, carry):
        c0 = 2 * pr
        drain(c0, 0)
        issue(c0 + 1, 1)
        compute(c0, 0)
        drain(c0 + 1, 1)

        @pl.when(pr + 1 < NCH // 2)
        def _():
            issue(c0 + 2, 0)

        compute(c0 + 1, 1)
        return carry

    lax.fori_loop(0, NCH // 2, pair_body, 0)
    pltpu.sync_copy(o_v, out_hbm.at[pl.ds(base, BPW)])


def kernel(user, item, user_factors, item_factors):
    return _sc_dot_kernel(user.astype(jnp.int32), item.astype(jnp.int32),
                          user_factors, item_factors)


# speedup vs baseline: 3.3389x; 1.0167x over previous
import functools

import jax
import jax.numpy as jnp
from jax import lax
from jax.experimental import pallas as pl
from jax.experimental.pallas import tpu as pltpu
from jax.experimental.pallas import tpu_sc as plsc

B = 16384
D = 128
NC = 2
NS = 16
NW = NC * NS
BPW = B // NW
CH = 128
NCH = BPW // CH

_mesh = plsc.VectorSubcoreMesh(core_axis_name="c", subcore_axis_name="s")


@functools.partial(
    pl.kernel,
    mesh=_mesh,
    compiler_params=pltpu.CompilerParams(needs_layout_passes=False),
    out_type=jax.ShapeDtypeStruct((B,), jnp.float32),
    scratch_types=[
        pltpu.VMEM((BPW,), jnp.int32),
        pltpu.VMEM((BPW,), jnp.int32),
        pltpu.VMEM((CH, D), jnp.float32),
        pltpu.VMEM((CH, D), jnp.float32),
        pltpu.VMEM((CH, D), jnp.float32),
        pltpu.VMEM((CH, D), jnp.float32),
        pltpu.VMEM((BPW,), jnp.float32),
        pltpu.SemaphoreType.DMA,
        pltpu.SemaphoreType.DMA,
        pltpu.SemaphoreType.DMA,
        pltpu.SemaphoreType.DMA,
        pltpu.SemaphoreType.DMA,
        pltpu.SemaphoreType.DMA,
    ],
)
def _sc_dot_kernel(user_hbm, item_hbm, uf_hbm, if_hbm, out_hbm,
                   uidx_v, iidx_v, u0_v, u1_v, v0_v, v1_v, o_v,
                   sem_u0, sem_u1, sem_v0, sem_v1, sem_i, sem_o):
    wid = lax.axis_index("s") * NC + lax.axis_index("c")
    base = wid * BPW
    u_bufs = (u0_v, u1_v)
    v_bufs = (v0_v, v1_v)
    u_sems = (sem_u0, sem_u1)
    v_sems = (sem_v0, sem_v1)
    lane = lax.iota(jnp.int32, 16)

    cpi_u = pltpu.async_copy(user_hbm.at[pl.ds(base, BPW)], uidx_v, sem_i)
    cpi_i = pltpu.async_copy(item_hbm.at[pl.ds(base, BPW)], iidx_v, sem_i)
    cpi_u.wait()
    cpi_i.wait()

    def issue(c, p):
        pltpu.async_copy(uf_hbm.at[uidx_v.at[pl.ds(c * CH, CH)]],
                         u_bufs[p], u_sems[p])
        pltpu.async_copy(if_hbm.at[iidx_v.at[pl.ds(c * CH, CH)]],
                         v_bufs[p], v_sems[p])

    def drain(c, p):
        pltpu.make_async_copy(uf_hbm.at[uidx_v.at[pl.ds(c * CH, CH)]],
                              u_bufs[p], u_sems[p]).wait()
        pltpu.make_async_copy(if_hbm.at[iidx_v.at[pl.ds(c * CH, CH)]],
                              v_bufs[p], v_sems[p]).wait()

    def compute(c, p):
        u_v, v_v = u_bufs[p], v_bufs[p]

        def group_body(g, carry):
            rows = g * 16 + lane

            zero = jnp.zeros((16,), jnp.float32)

            @plsc.parallel_loop(0, D, step=2, unroll=4,
                                carry=(zero, zero, lane))
            def acc_loop(d, state, rows=rows):
                a0, a1, col = state
                c1 = (col + 1) & (D - 1)
                a0 = a0 + plsc.load_gather(u_v, [rows, col]) * \
                    plsc.load_gather(v_v, [rows, col])
                a1 = a1 + plsc.load_gather(u_v, [rows, c1]) * \
                    plsc.load_gather(v_v, [rows, c1])
                return a0, a1, (col + 2) & (D - 1)

            o_v[pl.ds(c * CH + g * 16, 16)] = acc_loop[0] + acc_loop[1]
            return carry

        lax.fori_loop(0, CH // 16, group_body, 0)
        pltpu.async_copy(o_v.at[pl.ds(c * CH, CH)],
                         out_hbm.at[pl.ds(base + c * CH, CH)], sem_o)

    issue(0, 0)

    def pair_body(pr, carry):
        c0 = 2 * pr
        drain(c0, 0)
        issue(c0 + 1, 1)
        compute(c0, 0)
        drain(c0 + 1, 1)

        @pl.when(pr + 1 < NCH // 2)
        def _():
            issue(c0 + 2, 0)

        compute(c0 + 1, 1)
        return carry

    lax.fori_loop(0, NCH // 2, pair_body, 0)

    def drain_out(c, carry):
        pltpu.make_async_copy(o_v.at[pl.ds(c * CH, CH)],
                              out_hbm.at[pl.ds(base + c * CH, CH)],
                              sem_o).wait()
        return carry

    lax.fori_loop(0, NCH, drain_out, 0)


def kernel(user, item, user_factors, item_factors):
    return _sc_dot_kernel(user.astype(jnp.int32), item.astype(jnp.int32),
                          user_factors, item_factors)


# speedup vs baseline: 3.3769x; 1.0114x over previous
import functools

import jax
import jax.numpy as jnp
from jax import lax
from jax.experimental import pallas as pl
from jax.experimental.pallas import tpu as pltpu
from jax.experimental.pallas import tpu_sc as plsc

B = 16384
D = 128
NC = 2
NS = 16
NW = NC * NS
BPW = B // NW
CH = 128
NCH = BPW // CH

_mesh = plsc.VectorSubcoreMesh(core_axis_name="c", subcore_axis_name="s")


@functools.partial(
    pl.kernel,
    mesh=_mesh,
    compiler_params=pltpu.CompilerParams(needs_layout_passes=False),
    out_type=jax.ShapeDtypeStruct((B,), jnp.float32),
    scratch_types=[
        pltpu.VMEM((BPW,), jnp.int32),
        pltpu.VMEM((BPW,), jnp.int32),
        pltpu.VMEM((CH, D), jnp.float32),
        pltpu.VMEM((CH, D), jnp.float32),
        pltpu.VMEM((CH, D), jnp.float32),
        pltpu.VMEM((CH, D), jnp.float32),
        pltpu.VMEM((CH, D), jnp.float32),
        pltpu.VMEM((CH, D), jnp.float32),
        pltpu.VMEM((BPW,), jnp.float32),
        pltpu.SemaphoreType.DMA,
        pltpu.SemaphoreType.DMA,
        pltpu.SemaphoreType.DMA,
        pltpu.SemaphoreType.DMA,
        pltpu.SemaphoreType.DMA,
        pltpu.SemaphoreType.DMA,
        pltpu.SemaphoreType.DMA,
        pltpu.SemaphoreType.DMA,
    ],
)
def _sc_dot_kernel(user_hbm, item_hbm, uf_hbm, if_hbm, out_hbm,
                   uidx_v, iidx_v, u0_v, u1_v, u2_v, v0_v, v1_v, v2_v, o_v,
                   sem_u0, sem_u1, sem_u2, sem_v0, sem_v1, sem_v2,
                   sem_i, sem_o):
    wid = lax.axis_index("s") * NC + lax.axis_index("c")
    base = wid * BPW
    u_bufs = (u0_v, u1_v, u2_v)
    v_bufs = (v0_v, v1_v, v2_v)
    u_sems = (sem_u0, sem_u1, sem_u2)
    v_sems = (sem_v0, sem_v1, sem_v2)
    lane = lax.iota(jnp.int32, 16)

    cpi_u = pltpu.async_copy(user_hbm.at[pl.ds(base, BPW)], uidx_v, sem_i)
    cpi_i = pltpu.async_copy(item_hbm.at[pl.ds(base, BPW)], iidx_v, sem_i)
    cpi_u.wait()
    cpi_i.wait()

    def issue(c, p):
        pltpu.async_copy(uf_hbm.at[uidx_v.at[pl.ds(c * CH, CH)]],
                         u_bufs[p], u_sems[p])
        pltpu.async_copy(if_hbm.at[iidx_v.at[pl.ds(c * CH, CH)]],
                         v_bufs[p], v_sems[p])

    def drain(c, p):
        pltpu.make_async_copy(uf_hbm.at[uidx_v.at[pl.ds(c * CH, CH)]],
                              u_bufs[p], u_sems[p]).wait()
        pltpu.make_async_copy(if_hbm.at[iidx_v.at[pl.ds(c * CH, CH)]],
                              v_bufs[p], v_sems[p]).wait()

    def compute(c, p):
        u_v, v_v = u_bufs[p], v_bufs[p]

        def group_body(g, carry):
            rows = g * 16 + lane

            zero = jnp.zeros((16,), jnp.float32)

            @plsc.parallel_loop(0, D, step=2, unroll=4,
                                carry=(zero, zero, lane))
            def acc_loop(d, state, rows=rows):
                a0, a1, col = state
                c1 = (col + 1) & (D - 1)
                a0 = a0 + plsc.load_gather(u_v, [rows, col]) * \
                    plsc.load_gather(v_v, [rows, col])
                a1 = a1 + plsc.load_gather(u_v, [rows, c1]) * \
                    plsc.load_gather(v_v, [rows, c1])
                return a0, a1, (col + 2) & (D - 1)

            o_v[pl.ds(c * CH + g * 16, 16)] = acc_loop[0] + acc_loop[1]
            return carry

        lax.fori_loop(0, CH // 16, group_body, 0)
        pltpu.async_copy(o_v.at[pl.ds(c * CH, CH)],
                         out_hbm.at[pl.ds(base + c * CH, CH)], sem_o)

    issue(0, 0)
    issue(1, 1)
    for c in range(NCH):
        drain(c, c % 3)
        if c + 2 < NCH:
            issue(c + 2, (c + 2) % 3)
        compute(c, c % 3)

    def drain_out(c, carry):
        pltpu.make_async_copy(o_v.at[pl.ds(c * CH, CH)],
                              out_hbm.at[pl.ds(base + c * CH, CH)],
                              sem_o).wait()
        return carry

    lax.fori_loop(0, NCH, drain_out, 0)


def kernel(user, item, user_factors, item_factors):
    return _sc_dot_kernel(user.astype(jnp.int32), item.astype(jnp.int32),
                          user_factors, item_factors)


# speedup vs baseline: 3.4279x; 1.0151x over previous
import functools

import jax
import jax.numpy as jnp
from jax import lax
from jax.experimental import pallas as pl
from jax.experimental.pallas import tpu as pltpu
from jax.experimental.pallas import tpu_sc as plsc

B = 16384
D = 128
NC = 2
NS = 16
NW = NC * NS
BPW = B // NW
CH = 64
NCH = BPW // CH

_mesh = plsc.VectorSubcoreMesh(core_axis_name="c", subcore_axis_name="s")


@functools.partial(
    pl.kernel,
    mesh=_mesh,
    compiler_params=pltpu.CompilerParams(needs_layout_passes=False),
    out_type=jax.ShapeDtypeStruct((B,), jnp.float32),
    scratch_types=[
        pltpu.VMEM((BPW,), jnp.int32),
        pltpu.VMEM((BPW,), jnp.int32),
        pltpu.VMEM((CH, D), jnp.float32),
        pltpu.VMEM((CH, D), jnp.float32),
        pltpu.VMEM((CH, D), jnp.float32),
        pltpu.VMEM((CH, D), jnp.float32),
        pltpu.VMEM((CH, D), jnp.float32),
        pltpu.VMEM((CH, D), jnp.float32),
        pltpu.VMEM((CH, D), jnp.float32),
        pltpu.VMEM((CH, D), jnp.float32),
        pltpu.VMEM((BPW,), jnp.float32),
        pltpu.SemaphoreType.DMA,
        pltpu.SemaphoreType.DMA,
        pltpu.SemaphoreType.DMA,
        pltpu.SemaphoreType.DMA,
        pltpu.SemaphoreType.DMA,
        pltpu.SemaphoreType.DMA,
        pltpu.SemaphoreType.DMA,
        pltpu.SemaphoreType.DMA,
        pltpu.SemaphoreType.DMA,
        pltpu.SemaphoreType.DMA,
    ],
)
def _sc_dot_kernel(user_hbm, item_hbm, uf_hbm, if_hbm, out_hbm,
                   uidx_v, iidx_v, u0_v, u1_v, u2_v, u3_v,
                   v0_v, v1_v, v2_v, v3_v, o_v,
                   sem_u0, sem_u1, sem_u2, sem_u3,
                   sem_v0, sem_v1, sem_v2, sem_v3,
                   sem_i, sem_o):
    wid = lax.axis_index("s") * NC + lax.axis_index("c")
    base = wid * BPW
    u_bufs = (u0_v, u1_v, u2_v, u3_v)
    v_bufs = (v0_v, v1_v, v2_v, v3_v)
    u_sems = (sem_u0, sem_u1, sem_u2, sem_u3)
    v_sems = (sem_v0, sem_v1, sem_v2, sem_v3)
    lane = lax.iota(jnp.int32, 16)

    cpi_u = pltpu.async_copy(user_hbm.at[pl.ds(base, BPW)], uidx_v, sem_i)
    cpi_i = pltpu.async_copy(item_hbm.at[pl.ds(base, BPW)], iidx_v, sem_i)
    cpi_u.wait()
    cpi_i.wait()

    def issue(c, p):
        pltpu.async_copy(uf_hbm.at[uidx_v.at[pl.ds(c * CH, CH)]],
                         u_bufs[p], u_sems[p])
        pltpu.async_copy(if_hbm.at[iidx_v.at[pl.ds(c * CH, CH)]],
                         v_bufs[p], v_sems[p])

    def drain(c, p):
        pltpu.make_async_copy(uf_hbm.at[uidx_v.at[pl.ds(c * CH, CH)]],
                              u_bufs[p], u_sems[p]).wait()
        pltpu.make_async_copy(if_hbm.at[iidx_v.at[pl.ds(c * CH, CH)]],
                              v_bufs[p], v_sems[p]).wait()

    def compute(c, p):
        u_v, v_v = u_bufs[p], v_bufs[p]

        def group_body(g, carry):
            rows = g * 16 + lane

            zero = jnp.zeros((16,), jnp.float32)

            @plsc.parallel_loop(0, D, step=2, unroll=4,
                                carry=(zero, zero, lane))
            def acc_loop(d, state, rows=rows):
                a0, a1, col = state
                c1 = (col + 1) & (D - 1)
                a0 = a0 + plsc.load_gather(u_v, [rows, col]) * \
                    plsc.load_gather(v_v, [rows, col])
                a1 = a1 + plsc.load_gather(u_v, [rows, c1]) * \
                    plsc.load_gather(v_v, [rows, c1])
                return a0, a1, (col + 2) & (D - 1)

            o_v[pl.ds(c * CH + g * 16, 16)] = acc_loop[0] + acc_loop[1]
            return carry

        lax.fori_loop(0, CH // 16, group_body, 0)
        pltpu.async_copy(o_v.at[pl.ds(c * CH, CH)],
                         out_hbm.at[pl.ds(base + c * CH, CH)], sem_o)

    issue(0, 0)
    issue(1, 1)
    issue(2, 2)
    for c in range(NCH):
        drain(c, c % 4)
        if c + 3 < NCH:
            issue(c + 3, (c + 3) % 4)
        compute(c, c % 4)

    def drain_out(c, carry):
        pltpu.make_async_copy(o_v.at[pl.ds(c * CH, CH)],
                              out_hbm.at[pl.ds(base + c * CH, CH)],
                              sem_o).wait()
        return carry

    lax.fori_loop(0, NCH, drain_out, 0)


def kernel(user, item, user_factors, item_factors):
    return _sc_dot_kernel(user.astype(jnp.int32), item.astype(jnp.int32),
                          user_factors, item_factors)
